# initial kernel scaffold (unmeasured)
import jax
import jax.numpy as jnp
from jax import lax
from jax.experimental import pallas as pl
from jax.experimental.pallas import tpu as pltpu

N_DEV = 32
N_STEPS = N_DEV - 1
M = 2048
N = 2048
CHUNK = M // N_DEV


def kernel(A, B):
    def body(a_ref, b_ref, out_ref,
             rs_send, rs_recv, ag_buf,
             rs_send_sems, rs_recv_sems, ag_send_sems, ag_recv_sems):
        my = lax.axis_index("i")
        left = lax.rem(my + N_DEV - 1, N_DEV)
        right = lax.rem(my + 1, N_DEV)

        barrier_sem = pltpu.get_barrier_semaphore()
        for nbr in [left, right]:
            pl.semaphore_signal(
                barrier_sem, inc=1,
                device_id=(nbr,), device_id_type=pl.DeviceIdType.MESH,
            )
        pl.semaphore_wait(barrier_sem, 2)

        out_ref[...] = jnp.dot(
            a_ref[...], b_ref[...], preferred_element_type=jnp.float32
        )

        for s in range(N_STEPS):
            k = lax.rem(my + N_DEV - s, N_DEV)
            rows = pl.ds(k * CHUNK, CHUNK)
            part = out_ref[rows, :]
            if s == 0:
                val = part.astype(jnp.bfloat16)
            else:
                val = (part + rs_recv[s - 1].astype(jnp.float32)).astype(
                    jnp.bfloat16
                )
            rs_send[s] = val
            rdma = pltpu.make_async_remote_copy(
                src_ref=rs_send.at[s],
                dst_ref=rs_recv.at[s],
                send_sem=rs_send_sems.at[s],
                recv_sem=rs_recv_sems.at[s],
                device_id=(right,),
                device_id_type=pl.DeviceIdType.MESH,
            )
            rdma.start()
            rdma.wait()

        own = lax.rem(my + 1, N_DEV)
        own_rows = pl.ds(own * CHUNK, CHUNK)
        red = out_ref[own_rows, :] + rs_recv[N_STEPS - 1].astype(jnp.float32)
        out_ref[own_rows, :] = red
        ag_buf[own_rows, :] = red.astype(jnp.bfloat16)

        for t in range(N_STEPS):
            g = lax.rem(my + 1 + N_DEV - t, N_DEV)
            g_rows = pl.ds(g * CHUNK, CHUNK)
            rdma = pltpu.make_async_remote_copy(
                src_ref=ag_buf.at[g_rows, :],
                dst_ref=ag_buf.at[g_rows, :],
                send_sem=ag_send_sems.at[t],
                recv_sem=ag_recv_sems.at[t],
                device_id=(right,),
                device_id_type=pl.DeviceIdType.MESH,
            )
            rdma.start()
            rdma.wait()
            r = lax.rem(my + N_DEV - t, N_DEV)
            r_rows = pl.ds(r * CHUNK, CHUNK)
            out_ref[r_rows, :] = ag_buf[r_rows, :].astype(jnp.float32)

    return pl.pallas_call(
        body,
        out_shape=jax.ShapeDtypeStruct((M, N), jnp.float32),
        in_specs=[
            pl.BlockSpec(memory_space=pltpu.VMEM),
            pl.BlockSpec(memory_space=pltpu.VMEM),
        ],
        out_specs=pl.BlockSpec(memory_space=pltpu.VMEM),
        scratch_shapes=[
            pltpu.VMEM((N_STEPS, CHUNK, N), jnp.bfloat16),
            pltpu.VMEM((N_STEPS, CHUNK, N), jnp.bfloat16),
            pltpu.VMEM((M, N), jnp.bfloat16),
            pltpu.SemaphoreType.DMA((N_STEPS,)),
            pltpu.SemaphoreType.DMA((N_STEPS,)),
            pltpu.SemaphoreType.DMA((N_STEPS,)),
            pltpu.SemaphoreType.DMA((N_STEPS,)),
        ],
        compiler_params=pltpu.CompilerParams(collective_id=0),
    )(A.astype(jnp.bfloat16), B.astype(jnp.bfloat16))


# baseline (device time: 336525 ns/iter reference)
import jax
import jax.numpy as jnp
from jax import lax
from jax.experimental import pallas as pl
from jax.experimental.pallas import tpu as pltpu

N_DEV = 32
N_STEPS = N_DEV - 1
M = 2048
N = 2048
CHUNK = M // N_DEV


def kernel(A, B):
    def body(a_ref, b_ref, out_ref,
             rs_send, rs_recv, ag_buf,
             rs_send_sems, rs_recv_sems, ag_send_sems, ag_recv_sems):
        my = lax.axis_index("i")
        left = lax.rem(my + N_DEV - 1, N_DEV)
        right = lax.rem(my + 1, N_DEV)

        barrier_sem = pltpu.get_barrier_semaphore()
        for nbr in [left, right]:
            pl.semaphore_signal(
                barrier_sem, inc=1,
                device_id=(nbr,), device_id_type=pl.DeviceIdType.MESH,
            )
        pl.semaphore_wait(barrier_sem, 2)

        out_ref[...] = jnp.dot(
            a_ref[...], b_ref[...], preferred_element_type=jnp.float32
        )

        for s in range(N_STEPS):
            k = lax.rem(my + N_DEV - s, N_DEV)
            rows = pl.ds(k * CHUNK, CHUNK)
            part = out_ref[rows, :]
            if s == 0:
                val = part.astype(jnp.bfloat16)
            else:
                val = (part + rs_recv[s - 1].astype(jnp.float32)).astype(
                    jnp.bfloat16
                )
            rs_send[s] = val
            rdma = pltpu.make_async_remote_copy(
                src_ref=rs_send.at[s],
                dst_ref=rs_recv.at[s],
                send_sem=rs_send_sems.at[s],
                recv_sem=rs_recv_sems.at[s],
                device_id=(right,),
                device_id_type=pl.DeviceIdType.MESH,
            )
            rdma.start()
            rdma.wait()

        own = lax.rem(my + 1, N_DEV)
        own_rows = pl.ds(own * CHUNK, CHUNK)
        red = out_ref[own_rows, :] + rs_recv[N_STEPS - 1].astype(jnp.float32)
        out_ref[own_rows, :] = red
        ag_buf[own_rows, :] = red.astype(jnp.bfloat16)

        for t in range(N_STEPS):
            g = lax.rem(my + 1 + N_DEV - t, N_DEV)
            g_rows = pl.ds(g * CHUNK, CHUNK)
            rdma = pltpu.make_async_remote_copy(
                src_ref=ag_buf.at[g_rows, :],
                dst_ref=ag_buf.at[g_rows, :],
                send_sem=ag_send_sems.at[t],
                recv_sem=ag_recv_sems.at[t],
                device_id=(right,),
                device_id_type=pl.DeviceIdType.MESH,
            )
            rdma.start()
            rdma.wait()
            r = lax.rem(my + N_DEV - t, N_DEV)
            r_rows = pl.ds(r * CHUNK, CHUNK)
            out_ref[r_rows, :] = ag_buf[r_rows, :].astype(jnp.float32)

    return pl.pallas_call(
        body,
        out_shape=jax.ShapeDtypeStruct((M, N), jnp.float32),
        in_specs=[
            pl.BlockSpec(memory_space=pltpu.VMEM),
            pl.BlockSpec(memory_space=pltpu.VMEM),
        ],
        out_specs=pl.BlockSpec(memory_space=pltpu.VMEM),
        scratch_shapes=[
            pltpu.VMEM((N_STEPS, CHUNK, N), jnp.bfloat16),
            pltpu.VMEM((N_STEPS, CHUNK, N), jnp.bfloat16),
            pltpu.VMEM((M, N), jnp.bfloat16),
            pltpu.SemaphoreType.DMA((N_STEPS,)),
            pltpu.SemaphoreType.DMA((N_STEPS,)),
            pltpu.SemaphoreType.DMA((N_STEPS,)),
            pltpu.SemaphoreType.DMA((N_STEPS,)),
        ],
        compiler_params=pltpu.CompilerParams(
            collective_id=0, vmem_limit_bytes=60 * 1024 * 1024
        ),
    )(A.astype(jnp.bfloat16), B.astype(jnp.bfloat16))


# device time: 169400 ns/iter; 1.9866x vs baseline; 1.9866x over previous
import jax
import jax.numpy as jnp
from jax import lax
from jax.experimental import pallas as pl
from jax.experimental.pallas import tpu as pltpu

N_DEV = 32
M = 2048
N = 2048

COL_OFF = [0, 512, 1280]
COL_W = [512, 768, 768]
STREAM_DIMS = [
    [("x", 0), ("y", 0), ("y", 1), ("z", 0), ("z", 1)],
    [("y", 0), ("y", 1), ("z", 0), ("z", 1), ("x", 0)],
    [("z", 0), ("z", 1), ("x", 0), ("y", 0), ("y", 1)],
]
H = [1024, 512, 256, 128, 64]

ROUNDS_RS = [
    [(0, 0), (1, 0), (2, 0)],
    [(1, 1), (2, 1)],
    [(0, 1), (1, 2), (2, 2)],
    [(0, 2), (1, 3)],
    [(0, 3), (1, 4), (2, 3)],
    [(0, 4), (2, 4)],
]
ROUNDS_AG = [
    [(0, 4), (2, 4)],
    [(0, 3), (1, 4), (2, 3)],
    [(0, 2), (1, 3)],
    [(0, 1), (1, 2), (2, 2)],
    [(1, 1), (2, 1)],
    [(0, 0), (1, 0), (2, 0)],
]


def kernel(A, B):
    n_bufs = 3 * 5

    def body(a_ref, b_ref, out_ref, *scratch):
        send_bufs = scratch[0:n_bufs]
        rs_recv_bufs = scratch[n_bufs:2 * n_bufs]
        ag_recv_bufs = scratch[2 * n_bufs:3 * n_bufs]
        rs_send_sems, rs_recv_sems, ag_send_sems, ag_recv_sems = scratch[
            3 * n_bufs:
        ]

        def buf(bank, s, k):
            return bank[s * 5 + k]

        my = lax.axis_index("i")
        zc = my // 8
        rr = my % 8
        yc = rr // 2
        xc = jnp.where(yc % 2 == 0, rr % 2, 1 - rr % 2)

        def mesh_id(x, y, z):
            r = 2 * y + jnp.where(y % 2 == 0, x, 1 - x)
            return z * 8 + r

        def partner_of(dim, lvl):
            if dim == "x":
                return my ^ 1
            if dim == "y":
                return mesh_id(xc, yc ^ (1 << lvl), zc)
            return mesh_id(xc, yc, zc ^ (1 << lvl))

        def bit_of(dim, lvl):
            if dim == "x":
                return xc
            if dim == "y":
                return (yc >> lvl) & 1
            return (zc >> lvl) & 1

        partners = [
            partner_of("x", 0), partner_of("y", 0), partner_of("y", 1),
            partner_of("z", 0), partner_of("z", 1),
        ]
        barrier_sem = pltpu.get_barrier_semaphore()
        for p in partners:
            pl.semaphore_signal(
                barrier_sem, inc=1,
                device_id=(p,), device_id_type=pl.DeviceIdType.MESH,
            )
        pl.semaphore_wait(barrier_sem, len(partners))

        out_ref[...] = jnp.dot(
            a_ref[...], b_ref[...], preferred_element_type=jnp.float32
        )

        lo = [jnp.int32(0)] * 3

        for rnd in ROUNDS_RS:
            inflight = []
            for (s, k) in rnd:
                dim, lvl = STREAM_DIMS[s][k]
                b = bit_of(dim, lvl)
                h = H[k]
                cols = pl.ds(COL_OFF[s], COL_W[s])
                send_lo = lo[s] + (1 - b) * h
                keep_lo = lo[s] + b * h
                sb = buf(send_bufs, s, k)
                rb = buf(rs_recv_bufs, s, k)
                sb[...] = out_ref[pl.ds(send_lo, h), cols].astype(jnp.bfloat16)
                rdma = pltpu.make_async_remote_copy(
                    src_ref=sb,
                    dst_ref=rb,
                    send_sem=rs_send_sems.at[s, k],
                    recv_sem=rs_recv_sems.at[s, k],
                    device_id=(partner_of(dim, lvl),),
                    device_id_type=pl.DeviceIdType.MESH,
                )
                rdma.start()
                inflight.append((s, k, h, cols, keep_lo, rb, rdma))
            for (s, k, h, cols, keep_lo, rb, rdma) in inflight:
                rdma.wait()
                rows = pl.ds(keep_lo, h)
                out_ref[rows, cols] = (
                    out_ref[rows, cols] + rb[...].astype(jnp.float32)
                )
                lo[s] = keep_lo

        for rnd in ROUNDS_AG:
            inflight = []
            for (s, k) in rnd:
                dim, lvl = STREAM_DIMS[s][k]
                b = bit_of(dim, lvl)
                h = H[k]
                cols = pl.ds(COL_OFF[s], COL_W[s])
                sb = buf(send_bufs, s, k)
                rb = buf(ag_recv_bufs, s, k)
                sb[...] = out_ref[pl.ds(lo[s], h), cols].astype(jnp.bfloat16)
                rdma = pltpu.make_async_remote_copy(
                    src_ref=sb,
                    dst_ref=rb,
                    send_sem=ag_send_sems.at[s, k],
                    recv_sem=ag_recv_sems.at[s, k],
                    device_id=(partner_of(dim, lvl),),
                    device_id_type=pl.DeviceIdType.MESH,
                )
                rdma.start()
                parent_lo = lo[s] - b * h
                other_lo = parent_lo + (1 - b) * h
                inflight.append((s, h, cols, other_lo, rb, rdma))
                lo[s] = parent_lo
            for (s, h, cols, other_lo, rb, rdma) in inflight:
                rdma.wait()
                out_ref[pl.ds(other_lo, h), cols] = rb[...].astype(jnp.float32)

    scratch_shapes = []
    for s in range(3):
        for k in range(5):
            scratch_shapes.append(pltpu.VMEM((H[k], COL_W[s]), jnp.bfloat16))
    for s in range(3):
        for k in range(5):
            scratch_shapes.append(pltpu.VMEM((H[k], COL_W[s]), jnp.bfloat16))
    for s in range(3):
        for k in range(5):
            scratch_shapes.append(pltpu.VMEM((H[k], COL_W[s]), jnp.bfloat16))
    scratch_shapes += [
        pltpu.SemaphoreType.DMA((3, 5)),
        pltpu.SemaphoreType.DMA((3, 5)),
        pltpu.SemaphoreType.DMA((3, 5)),
        pltpu.SemaphoreType.DMA((3, 5)),
    ]

    return pl.pallas_call(
        body,
        out_shape=jax.ShapeDtypeStruct((M, N), jnp.float32),
        in_specs=[
            pl.BlockSpec(memory_space=pltpu.VMEM),
            pl.BlockSpec(memory_space=pltpu.VMEM),
        ],
        out_specs=pl.BlockSpec(memory_space=pltpu.VMEM),
        scratch_shapes=scratch_shapes,
        compiler_params=pltpu.CompilerParams(
            collective_id=0, vmem_limit_bytes=60 * 1024 * 1024
        ),
    )(A.astype(jnp.bfloat16), B.astype(jnp.bfloat16))


# device time: 159268 ns/iter; 2.1129x vs baseline; 1.0636x over previous
import jax
import jax.numpy as jnp
from jax import lax
from jax.experimental import pallas as pl
from jax.experimental.pallas import tpu as pltpu

N_DEV = 32
M = 2048
N = 2048

COL_OFF = [0, 512, 1280]
COL_W = [512, 768, 768]
STREAM_DIMS = [
    [("x", 0), ("y", 0), ("y", 1), ("z", 0), ("z", 1)],
    [("y", 0), ("y", 1), ("z", 0), ("z", 1), ("x", 0)],
    [("z", 0), ("z", 1), ("x", 0), ("y", 0), ("y", 1)],
]
H = [1024, 512, 256, 128, 64]

ROUNDS_RS = [
    [(1, 0), (2, 0), (0, 0)],
    [(1, 1), (2, 1)],
    [(0, 1), (1, 2), (2, 2)],
    [(0, 2), (1, 3)],
    [(0, 3), (1, 4), (2, 3)],
    [(0, 4), (2, 4)],
]
ROUNDS_AG = [
    [(0, 4), (2, 4)],
    [(0, 3), (1, 4), (2, 3)],
    [(0, 2), (1, 3)],
    [(0, 1), (1, 2), (2, 2)],
    [(1, 1), (2, 1)],
    [(0, 0), (1, 0), (2, 0)],
]


def kernel(A, B):
    n_bufs = 3 * 5

    def body(a_ref, b_ref, out_ref, acc_ref, *scratch):
        rs_recv_bufs = scratch[0:n_bufs]
        rs_send_sems, rs_recv_sems, ag_send_sems, ag_recv_sems = scratch[
            n_bufs:
        ]

        my = lax.axis_index("i")
        zc = my // 8
        rr = my % 8
        yc = rr // 2
        xc = jnp.where(yc % 2 == 0, rr % 2, 1 - rr % 2)

        def mesh_id(x, y, z):
            r = 2 * y + jnp.where(y % 2 == 0, x, 1 - x)
            return z * 8 + r

        def partner_of(dim, lvl):
            if dim == "x":
                return my ^ 1
            if dim == "y":
                return mesh_id(xc, yc ^ (1 << lvl), zc)
            return mesh_id(xc, yc, zc ^ (1 << lvl))

        def bit_of(dim, lvl):
            if dim == "x":
                return xc
            if dim == "y":
                return (yc >> lvl) & 1
            return (zc >> lvl) & 1

        partners = [
            partner_of("x", 0), partner_of("y", 0), partner_of("y", 1),
            partner_of("z", 0), partner_of("z", 1),
        ]
        barrier_sem = pltpu.get_barrier_semaphore()
        for p in partners:
            pl.semaphore_signal(
                barrier_sem, inc=1,
                device_id=(p,), device_id_type=pl.DeviceIdType.MESH,
            )
        pl.semaphore_wait(barrier_sem, len(partners))

        lo = [jnp.int32(0)] * 3

        def col_ds(s):
            return pl.ds(COL_OFF[s], COL_W[s])

        def start_rs(s, k):
            dim, lvl = STREAM_DIMS[s][k]
            b = bit_of(dim, lvl)
            h = H[k]
            send_lo = lo[s] + (1 - b) * h
            rdma = pltpu.make_async_remote_copy(
                src_ref=acc_ref.at[pl.ds(send_lo, h), col_ds(s)],
                dst_ref=rs_recv_bufs[s * 5 + k],
                send_sem=rs_send_sems.at[s, k],
                recv_sem=rs_recv_sems.at[s, k],
                device_id=(partner_of(dim, lvl),),
                device_id_type=pl.DeviceIdType.MESH,
            )
            rdma.start()
            return (k, rdma)

        def finish_rs(s, pending):
            k, rdma = pending
            dim, lvl = STREAM_DIMS[s][k]
            b = bit_of(dim, lvl)
            h = H[k]
            keep_lo = lo[s] + b * h
            rdma.wait()
            rows = pl.ds(keep_lo, h)
            acc_ref[rows, col_ds(s)] = (
                acc_ref[rows, col_ds(s)] + rs_recv_bufs[s * 5 + k][...]
            )
            lo[s] = keep_lo

        pending = {}
        for (s, k) in ROUNDS_RS[0]:
            acc_ref[:, col_ds(s)] = jnp.dot(
                a_ref[...],
                b_ref[:, col_ds(s)],
                preferred_element_type=jnp.float32,
            ).astype(jnp.bfloat16)
            pending[s] = start_rs(s, k)
        for rnd in ROUNDS_RS[1:]:
            for (s, k) in rnd:
                finish_rs(s, pending[s])
                pending[s] = start_rs(s, k)
        for s in range(3):
            finish_rs(s, pending[s])

        def start_ag(s, k):
            dim, lvl = STREAM_DIMS[s][k]
            h = H[k]
            rdma = pltpu.make_async_remote_copy(
                src_ref=acc_ref.at[pl.ds(lo[s], h), col_ds(s)],
                dst_ref=acc_ref.at[pl.ds(lo[s], h), col_ds(s)],
                send_sem=ag_send_sems.at[s, k],
                recv_sem=ag_recv_sems.at[s, k],
                device_id=(partner_of(dim, lvl),),
                device_id_type=pl.DeviceIdType.MESH,
            )
            rdma.start()
            return (k, rdma)

        def finish_ag(s, pending_item):
            k, rdma = pending_item
            dim, lvl = STREAM_DIMS[s][k]
            b = bit_of(dim, lvl)
            rdma.wait()
            lo[s] = lo[s] - b * H[k]

        pending = {}
        for rnd in ROUNDS_AG:
            for (s, k) in rnd:
                if s in pending:
                    finish_ag(s, pending[s])
                pending[s] = start_ag(s, k)
        for s in range(3):
            finish_ag(s, pending[s])

        out_ref[...] = acc_ref[...].astype(jnp.float32)

    scratch_shapes = [pltpu.VMEM((M, N), jnp.bfloat16)]
    for s in range(3):
        for k in range(5):
            scratch_shapes.append(pltpu.VMEM((H[k], COL_W[s]), jnp.bfloat16))
    scratch_shapes += [
        pltpu.SemaphoreType.DMA((3, 5)),
        pltpu.SemaphoreType.DMA((3, 5)),
        pltpu.SemaphoreType.DMA((3, 5)),
        pltpu.SemaphoreType.DMA((3, 5)),
    ]

    def wrapped_body(a_ref, b_ref, out_ref, *scratch):
        return body(a_ref, b_ref, out_ref, scratch[0], *scratch[1:])

    return pl.pallas_call(
        wrapped_body,
        out_shape=jax.ShapeDtypeStruct((M, N), jnp.float32),
        in_specs=[
            pl.BlockSpec(memory_space=pltpu.VMEM),
            pl.BlockSpec(memory_space=pltpu.VMEM),
        ],
        out_specs=pl.BlockSpec(memory_space=pltpu.VMEM),
        scratch_shapes=scratch_shapes,
        compiler_params=pltpu.CompilerParams(
            collective_id=0, vmem_limit_bytes=60 * 1024 * 1024
        ),
    )(A.astype(jnp.bfloat16), B.astype(jnp.bfloat16))


# device time: 142552 ns/iter; 2.3607x vs baseline; 1.1173x over previous
import jax
import jax.numpy as jnp
from jax import lax
from jax.experimental import pallas as pl
from jax.experimental.pallas import tpu as pltpu

N_DEV = 32
M = 2048
N = 2048

MAJOR_COLS = {0: (0, 512), 1: (512, 768), 2: (1280, 768)}
COL_OFF = [0, 384, 512, 1024, 1280, 1792]
COL_W = [384, 128, 512, 256, 512, 256]
STREAM_DIMS = [
    [("x", 0), ("y", 0), ("y", 1), ("z", 0), ("z", 1)],
    [("x", 0), ("y", 1), ("y", 0), ("z", 1), ("z", 0)],
    [("y", 0), ("y", 1), ("z", 0), ("z", 1), ("x", 0)],
    [("y", 1), ("y", 0), ("z", 1), ("z", 0), ("x", 0)],
    [("z", 0), ("z", 1), ("x", 0), ("y", 0), ("y", 1)],
    [("z", 1), ("z", 0), ("x", 0), ("y", 1), ("y", 0)],
]
H = [1024, 512, 256, 128, 64]

_SKEL_RS = [
    [(1, 0), (2, 0), (0, 0)],
    [(1, 1), (2, 1)],
    [(0, 1), (1, 2), (2, 2)],
    [(0, 2), (1, 3)],
    [(0, 3), (1, 4), (2, 3)],
    [(0, 4), (2, 4)],
]


def _expand(skel):
    return [[(2 * o + a, k) for (o, k) in rnd for a in (0, 1)] for rnd in skel]


ROUNDS_RS = _expand(_SKEL_RS)
ROUNDS_AG = _expand([
    [(0, 4), (2, 4)],
    [(0, 3), (1, 4), (2, 3)],
    [(0, 2), (1, 3)],
    [(0, 1), (1, 2), (2, 2)],
    [(1, 1), (2, 1)],
    [(0, 0), (1, 0), (2, 0)],
])

N_STREAMS = 6
N_STEPS = 5


def kernel(A, B):
    n_bufs = N_STREAMS * N_STEPS

    def body(a_ref, b_ref, out_ref, acc_ref, *scratch):
        rs_recv_bufs = scratch[0:n_bufs]
        rs_send_sems, rs_recv_sems, ag_send_sems, ag_recv_sems = scratch[
            n_bufs:
        ]

        my = lax.axis_index("i")
        zc = my // 8
        rr = my % 8
        yc = rr // 2
        xc = jnp.where(yc % 2 == 0, rr % 2, 1 - rr % 2)

        def mesh_id(x, y, z):
            r = 2 * y + jnp.where(y % 2 == 0, x, 1 - x)
            return z * 8 + r

        def partner_of(dim, lvl):
            if dim == "x":
                return my ^ 1
            if dim == "y":
                return mesh_id(xc, yc ^ (1 << lvl), zc)
            return mesh_id(xc, yc, zc ^ (1 << lvl))

        def bit_of(dim, lvl):
            if dim == "x":
                return xc
            if dim == "y":
                return (yc >> lvl) & 1
            return (zc >> lvl) & 1

        partners = [
            partner_of("x", 0), partner_of("y", 0), partner_of("y", 1),
            partner_of("z", 0), partner_of("z", 1),
        ]
        barrier_sem = pltpu.get_barrier_semaphore()
        for p in partners:
            pl.semaphore_signal(
                barrier_sem, inc=1,
                device_id=(p,), device_id_type=pl.DeviceIdType.MESH,
            )
        pl.semaphore_wait(barrier_sem, len(partners))

        lo = [jnp.int32(0)] * N_STREAMS

        def col_ds(s):
            return pl.ds(COL_OFF[s], COL_W[s])

        def start_rs(s, k):
            dim, lvl = STREAM_DIMS[s][k]
            b = bit_of(dim, lvl)
            h = H[k]
            send_lo = lo[s] + (1 - b) * h
            rdma = pltpu.make_async_remote_copy(
                src_ref=acc_ref.at[pl.ds(send_lo, h), col_ds(s)],
                dst_ref=rs_recv_bufs[s * N_STEPS + k],
                send_sem=rs_send_sems.at[s, k],
                recv_sem=rs_recv_sems.at[s, k],
                device_id=(partner_of(dim, lvl),),
                device_id_type=pl.DeviceIdType.MESH,
            )
            rdma.start()
            return (k, rdma)

        def finish_rs(s, pending_item):
            k, rdma = pending_item
            dim, lvl = STREAM_DIMS[s][k]
            b = bit_of(dim, lvl)
            h = H[k]
            keep_lo = lo[s] + b * h
            rdma.wait()
            rows = pl.ds(keep_lo, h)
            acc_ref[rows, col_ds(s)] = (
                acc_ref[rows, col_ds(s)] + rs_recv_bufs[s * N_STEPS + k][...]
            )
            lo[s] = keep_lo

        pending = {}
        for (o, k0) in _SKEL_RS[0]:
            off, w = MAJOR_COLS[o]
            acc_ref[:, pl.ds(off, w)] = jnp.dot(
                a_ref[...],
                b_ref[:, pl.ds(off, w)],
                preferred_element_type=jnp.float32,
            ).astype(jnp.bfloat16)
            for a in (0, 1):
                s = 2 * o + a
                pending[s] = start_rs(s, k0)
        for rnd in ROUNDS_RS[1:]:
            for (s, k) in rnd:
                finish_rs(s, pending[s])
                pending[s] = start_rs(s, k)
        for s in range(N_STREAMS):
            finish_rs(s, pending[s])

        def start_ag(s, k):
            dim, lvl = STREAM_DIMS[s][k]
            h = H[k]
            rdma = pltpu.make_async_remote_copy(
                src_ref=acc_ref.at[pl.ds(lo[s], h), col_ds(s)],
                dst_ref=acc_ref.at[pl.ds(lo[s], h), col_ds(s)],
                send_sem=ag_send_sems.at[s, k],
                recv_sem=ag_recv_sems.at[s, k],
                device_id=(partner_of(dim, lvl),),
                device_id_type=pl.DeviceIdType.MESH,
            )
            rdma.start()
            return (k, rdma)

        def finish_ag(s, pending_item):
            k, rdma = pending_item
            dim, lvl = STREAM_DIMS[s][k]
            b = bit_of(dim, lvl)
            rdma.wait()
            lo[s] = lo[s] - b * H[k]

        pending = {}
        for rnd in ROUNDS_AG:
            for (s, k) in rnd:
                if s in pending:
                    finish_ag(s, pending[s])
                pending[s] = start_ag(s, k)
        for s in range(N_STREAMS):
            finish_ag(s, pending[s])

        out_ref[...] = acc_ref[...].astype(jnp.float32)

    scratch_shapes = [pltpu.VMEM((M, N), jnp.bfloat16)]
    for s in range(N_STREAMS):
        for k in range(N_STEPS):
            scratch_shapes.append(pltpu.VMEM((H[k], COL_W[s]), jnp.bfloat16))
    scratch_shapes += [
        pltpu.SemaphoreType.DMA((N_STREAMS, N_STEPS)),
        pltpu.SemaphoreType.DMA((N_STREAMS, N_STEPS)),
        pltpu.SemaphoreType.DMA((N_STREAMS, N_STEPS)),
        pltpu.SemaphoreType.DMA((N_STREAMS, N_STEPS)),
    ]

    def wrapped_body(a_ref, b_ref, out_ref, *scratch):
        return body(a_ref, b_ref, out_ref, scratch[0], *scratch[1:])

    return pl.pallas_call(
        wrapped_body,
        out_shape=jax.ShapeDtypeStruct((M, N), jnp.float32),
        in_specs=[
            pl.BlockSpec(memory_space=pltpu.VMEM),
            pl.BlockSpec(memory_space=pltpu.VMEM),
        ],
        out_specs=pl.BlockSpec(memory_space=pltpu.VMEM),
        scratch_shapes=scratch_shapes,
        compiler_params=pltpu.CompilerParams(
            collective_id=0, vmem_limit_bytes=60 * 1024 * 1024
        ),
    )(A.astype(jnp.bfloat16), B.astype(jnp.bfloat16))


# device time: 128860 ns/iter; 2.6116x vs baseline; 1.1063x over previous
import jax
import jax.numpy as jnp
from jax import lax
from jax.experimental import pallas as pl
from jax.experimental.pallas import tpu as pltpu

N_DEV = 32
M = 2048
N = 2048

MAJOR_COLS = {0: (0, 512), 1: (512, 768), 2: (1280, 768)}
COL_OFF = [0, 384, 512, 1024, 1280, 1792]
COL_W = [384, 128, 512, 256, 512, 256]
STREAM_DIMS = [
    [("x", 0), ("y", 0), ("y", 1), ("z", 0), ("z", 1)],
    [("x", 0), ("y", 1), ("y", 0), ("z", 1), ("z", 0)],
    [("y", 0), ("y", 1), ("z", 0), ("z", 1), ("x", 0)],
    [("y", 1), ("y", 0), ("z", 1), ("z", 0), ("x", 0)],
    [("z", 0), ("z", 1), ("x", 0), ("y", 0), ("y", 1)],
    [("z", 1), ("z", 0), ("x", 0), ("y", 1), ("y", 0)],
]
H = [1024, 512, 256, 128, 64]

_SKEL_RS = [
    [(1, 0), (2, 0), (0, 0)],
    [(1, 1), (2, 1)],
    [(0, 1), (1, 2), (2, 2)],
    [(0, 2), (1, 3)],
    [(0, 3), (1, 4), (2, 3)],
    [(0, 4), (2, 4)],
]


def _expand(skel):
    return [[(2 * o + a, k) for (o, k) in rnd for a in (0, 1)] for rnd in skel]


ROUNDS_RS = _expand(_SKEL_RS)
ROUNDS_AG = _expand([
    [(0, 4), (2, 4)],
    [(0, 3), (1, 4), (2, 3)],
    [(0, 2), (1, 3)],
    [(0, 1), (1, 2), (2, 2)],
    [(1, 1), (2, 1)],
    [(0, 0), (1, 0), (2, 0)],
])

N_STREAMS = 6
N_STEPS = 5


def kernel(A, B):
    n_bufs = N_STREAMS * N_STEPS

    def body(a_ref, b_ref, out_ref, *scratch):
        rs_recv_bufs = scratch[0:n_bufs]
        rs_send_sems, rs_recv_sems, ag_send_sems, ag_recv_sems = scratch[
            n_bufs:
        ]

        my = lax.axis_index("i")
        zc = my // 8
        rr = my % 8
        yc = rr // 2
        xc = jnp.where(yc % 2 == 0, rr % 2, 1 - rr % 2)

        def mesh_id(x, y, z):
            r = 2 * y + jnp.where(y % 2 == 0, x, 1 - x)
            return z * 8 + r

        def partner_of(dim, lvl):
            if dim == "x":
                return my ^ 1
            if dim == "y":
                return mesh_id(xc, yc ^ (1 << lvl), zc)
            return mesh_id(xc, yc, zc ^ (1 << lvl))

        def bit_of(dim, lvl):
            if dim == "x":
                return xc
            if dim == "y":
                return (yc >> lvl) & 1
            return (zc >> lvl) & 1

        partners = [
            partner_of("x", 0), partner_of("y", 0), partner_of("y", 1),
            partner_of("z", 0), partner_of("z", 1),
        ]
        barrier_sem = pltpu.get_barrier_semaphore()
        for p in partners:
            pl.semaphore_signal(
                barrier_sem, inc=1,
                device_id=(p,), device_id_type=pl.DeviceIdType.MESH,
            )
        pl.semaphore_wait(barrier_sem, len(partners))

        lo = [jnp.int32(0)] * N_STREAMS

        def col_ds(s):
            return pl.ds(COL_OFF[s], COL_W[s])

        def start_rs(s, k):
            dim, lvl = STREAM_DIMS[s][k]
            b = bit_of(dim, lvl)
            h = H[k]
            send_lo = lo[s] + (1 - b) * h
            rdma = pltpu.make_async_remote_copy(
                src_ref=out_ref.at[pl.ds(send_lo, h), col_ds(s)],
                dst_ref=rs_recv_bufs[s * N_STEPS + k],
                send_sem=rs_send_sems.at[s, k],
                recv_sem=rs_recv_sems.at[s, k],
                device_id=(partner_of(dim, lvl),),
                device_id_type=pl.DeviceIdType.MESH,
            )
            rdma.start()
            return (k, rdma)

        def finish_rs(s, pending_item):
            k, rdma = pending_item
            dim, lvl = STREAM_DIMS[s][k]
            b = bit_of(dim, lvl)
            h = H[k]
            keep_lo = lo[s] + b * h
            rdma.wait()
            rows = pl.ds(keep_lo, h)
            out_ref[rows, col_ds(s)] = (
                out_ref[rows, col_ds(s)] + rs_recv_bufs[s * N_STEPS + k][...]
            )
            lo[s] = keep_lo

        def mm_half(s, row_lo):
            out_ref[pl.ds(row_lo, 1024), col_ds(s)] = jnp.dot(
                a_ref[pl.ds(row_lo, 1024), :],
                b_ref[:, col_ds(s)],
                preferred_element_type=jnp.float32,
            ).astype(jnp.bfloat16)

        pending = {}
        keep_los = {}
        for (o, k0) in _SKEL_RS[0]:
            for a in (0, 1):
                s = 2 * o + a
                dim, lvl = STREAM_DIMS[s][0]
                b = bit_of(dim, lvl)
                mm_half(s, (1 - b) * 1024)
                pending[s] = start_rs(s, k0)
                keep_los[s] = b * 1024
        for (o, k0) in _SKEL_RS[0]:
            for a in (0, 1):
                mm_half(2 * o + a, keep_los[2 * o + a])
        for rnd in ROUNDS_RS[1:]:
            for (s, k) in rnd:
                finish_rs(s, pending[s])
                pending[s] = start_rs(s, k)
        for s in range(N_STREAMS):
            finish_rs(s, pending[s])

        def start_ag(s, k):
            dim, lvl = STREAM_DIMS[s][k]
            h = H[k]
            rdma = pltpu.make_async_remote_copy(
                src_ref=out_ref.at[pl.ds(lo[s], h), col_ds(s)],
                dst_ref=out_ref.at[pl.ds(lo[s], h), col_ds(s)],
                send_sem=ag_send_sems.at[s, k],
                recv_sem=ag_recv_sems.at[s, k],
                device_id=(partner_of(dim, lvl),),
                device_id_type=pl.DeviceIdType.MESH,
            )
            rdma.start()
            return (k, rdma)

        def finish_ag(s, pending_item):
            k, rdma = pending_item
            dim, lvl = STREAM_DIMS[s][k]
            b = bit_of(dim, lvl)
            rdma.wait()
            lo[s] = lo[s] - b * H[k]

        pending = {}
        for rnd in ROUNDS_AG:
            for (s, k) in rnd:
                if s in pending:
                    finish_ag(s, pending[s])
                pending[s] = start_ag(s, k)
        for s in range(N_STREAMS):
            finish_ag(s, pending[s])


    scratch_shapes = []
    for s in range(N_STREAMS):
        for k in range(N_STEPS):
            scratch_shapes.append(pltpu.VMEM((H[k], COL_W[s]), jnp.bfloat16))
    scratch_shapes += [
        pltpu.SemaphoreType.DMA((N_STREAMS, N_STEPS)),
        pltpu.SemaphoreType.DMA((N_STREAMS, N_STEPS)),
        pltpu.SemaphoreType.DMA((N_STREAMS, N_STEPS)),
        pltpu.SemaphoreType.DMA((N_STREAMS, N_STEPS)),
    ]

    return pl.pallas_call(
        body,
        out_shape=jax.ShapeDtypeStruct((M, N), jnp.bfloat16),
        in_specs=[
            pl.BlockSpec(memory_space=pltpu.VMEM),
            pl.BlockSpec(memory_space=pltpu.VMEM),
        ],
        out_specs=pl.BlockSpec(memory_space=pltpu.VMEM),
        scratch_shapes=scratch_shapes,
        compiler_params=pltpu.CompilerParams(
            collective_id=0, vmem_limit_bytes=60 * 1024 * 1024
        ),
    )(A.astype(jnp.bfloat16), B.astype(jnp.bfloat16))
